# baseline (speedup 1.0000x reference)
import functools

import jax
import jax.numpy as jnp
from jax import lax
from jax.experimental import pallas as pl
from jax.experimental.pallas import tpu as pltpu
from jax.experimental.pallas import tpu_sc as plsc

B = 16384
D = 16
H1 = 64
H2 = 32
RPB = 8
CH = 128


@functools.cache
def _sc_gather():
    info = plsc.get_sparse_core_info()
    nw = info.num_cores * info.num_subcores
    bpw = B // nw
    ng = bpw // 16
    mesh = plsc.VectorSubcoreMesh(core_axis_name="c", subcore_axis_name="s")

    @functools.partial(
        pl.kernel,
        out_type=(
            jax.ShapeDtypeStruct((B, D), jnp.float32),
            jax.ShapeDtypeStruct((B, D), jnp.float32),
        ),
        mesh=mesh,
        compiler_params=pltpu.CompilerParams(use_tc_tiling_on_sc=True,
                                             needs_layout_passes=False),
        scratch_types=[
            pltpu.VMEM((bpw,), jnp.int32),
            pltpu.VMEM((bpw,), jnp.int32),
            pltpu.VMEM((CH,), jnp.int32),
            pltpu.VMEM((CH, RPB * D), jnp.float32),
            pltpu.VMEM((bpw, D), jnp.float32),
            pltpu.SemaphoreType.DMA,
        ],
    )
    def gather_kernel(uid_hbm, iid_hbm, utab_hbm, itab_hbm, uout_hbm, iout_hbm,
                      uidx_v, iidx_v, blk_v, blocks_v, out_v, sem):
        wid = lax.axis_index("s") * info.num_cores + lax.axis_index("c")
        base = wid * bpw
        pltpu.sync_copy(uid_hbm.at[pl.ds(base, bpw)], uidx_v)
        pltpu.sync_copy(iid_hbm.at[pl.ds(base, bpw)], iidx_v)
        lanes = lax.iota(jnp.int32, 16)

        def one_table(idx_v, tab_hbm, out_hbm):
            for c in range(bpw // CH):
                def compute_blk(g, _):
                    idx16 = idx_v[pl.ds(c * CH + g * 16, 16)]
                    blk_v[pl.ds(g * 16, 16)] = lax.shift_right_logical(idx16, 3)
                    return 0

                lax.fori_loop(0, CH // 16, compute_blk, 0)
                pltpu.async_copy(tab_hbm.at[blk_v], blocks_v, sem).wait()

                def extract(g, _):
                    idx16 = idx_v[pl.ds(c * CH + g * 16, 16)]
                    src_base = lax.shift_left(
                        lax.bitwise_and(idx16, jnp.int32(RPB - 1)), 4)
                    row16 = g * 16 + lanes
                    orow16 = c * CH + g * 16 + lanes
                    for j in range(D):
                        val = plsc.load_gather(blocks_v, [row16, src_base + j])
                        plsc.store_scatter(out_v, [orow16, lanes * 0 + j], val)
                    return 0

                lax.fori_loop(0, CH // 16, extract, 0)
            pltpu.sync_copy(out_v, out_hbm.at[pl.ds(base, bpw)])

        one_table(uidx_v, utab_hbm, uout_hbm)
        one_table(iidx_v, itab_hbm, iout_hbm)

    return gather_kernel


_BB = 2048


def _tc_body(u_ref, i_ref, w1u_ref, w1i_ref, b1_ref, w2_ref, b2_ref, w3_ref,
             c0_ref, out_ref):
    u = u_ref[...]
    it = i_ref[...]
    inter = jnp.sum(u * it, axis=1)
    h1 = jnp.dot(u, w1u_ref[...], preferred_element_type=jnp.float32)
    h1 = h1 + jnp.dot(it, w1i_ref[...], preferred_element_type=jnp.float32)
    h1 = jnp.maximum(h1 + b1_ref[...], 0.0)
    h2 = jnp.dot(h1, w2_ref[...], preferred_element_type=jnp.float32)
    h2 = jnp.maximum(h2 + b2_ref[...], 0.0)
    deep = jnp.sum(h2 * w3_ref[...], axis=1)
    out_ref[...] = inter + deep + c0_ref[0]


def _tc_mlp(u_emb, i_emb, w1u, w1i, b1, w2, b2, w3row, c0):
    rep = lambda shape: pl.BlockSpec(shape, lambda i: (0,) * len(shape))
    return pl.pallas_call(
        _tc_body,
        grid=(B // _BB,),
        in_specs=[
            pl.BlockSpec((_BB, D), lambda i: (i, 0)),
            pl.BlockSpec((_BB, D), lambda i: (i, 0)),
            rep((D, H1)),
            rep((D, H1)),
            rep((1, H1)),
            rep((H1, H2)),
            rep((1, H2)),
            rep((1, H2)),
            pl.BlockSpec(memory_space=pltpu.SMEM),
        ],
        out_specs=pl.BlockSpec((_BB,), lambda i: (i,)),
        out_shape=jax.ShapeDtypeStruct((B,), jnp.float32),
    )(u_emb, i_emb, w1u, w1i, b1, w2, b2, w3row, c0)


def kernel(user_id, item_id, user_table, item_table, fm_bias, W1, b1, W2, b2,
           W3, b3):
    uid = user_id.astype(jnp.int32)
    iid = item_id.astype(jnp.int32)
    utab128 = user_table.reshape(-1, RPB * D)
    itab128 = item_table.reshape(-1, RPB * D)
    u_emb, i_emb = _sc_gather()(uid, iid, utab128, itab128)
    c0 = fm_bias + b3
    return _tc_mlp(u_emb, i_emb, W1[:D], W1[D:], b1.reshape(1, H1), W2,
                   b2.reshape(1, H2), W3.reshape(1, H2), c0)


# speedup vs baseline: 4.9392x; 4.9392x over previous
import functools

import jax
import jax.numpy as jnp
from jax import lax
from jax.experimental import pallas as pl
from jax.experimental.pallas import tpu as pltpu
from jax.experimental.pallas import tpu_sc as plsc

B = 16384
D = 16
H1 = 64
H2 = 32


@functools.cache
def _sc_gather():
    info = plsc.get_sparse_core_info()
    nw = info.num_cores * info.num_subcores
    bpw = B // nw
    mesh = plsc.VectorSubcoreMesh(core_axis_name="c", subcore_axis_name="s")

    @functools.partial(
        pl.kernel,
        out_type=(
            jax.ShapeDtypeStruct((D, B), jnp.float32),
            jax.ShapeDtypeStruct((D, B), jnp.float32),
        ),
        mesh=mesh,
        compiler_params=pltpu.CompilerParams(use_tc_tiling_on_sc=True,
                                             needs_layout_passes=False),
        scratch_types=[
            pltpu.VMEM((bpw,), jnp.int32),
            pltpu.VMEM((bpw,), jnp.int32),
            pltpu.VMEM((D, 8 * 128), jnp.float32),
            pltpu.VMEM((D, 8 * 128), jnp.float32),
            pltpu.VMEM((D, bpw), jnp.float32),
            pltpu.VMEM((D, bpw), jnp.float32),
            pltpu.SemaphoreType.DMA,
            pltpu.SemaphoreType.DMA,
        ],
    )
    def gather_kernel(uid_hbm, iid_hbm, utab_hbm, itab_hbm, uout_hbm, iout_hbm,
                      uidx_v, iidx_v, ubuf_v, ibuf_v, urows_v, irows_v,
                      usem, isem):
        wid = lax.axis_index("s") * info.num_cores + lax.axis_index("c")
        base = wid * bpw
        pltpu.sync_copy(uid_hbm.at[pl.ds(base, bpw)], uidx_v)
        pltpu.sync_copy(iid_hbm.at[pl.ds(base, bpw)], iidx_v)
        lanes = lax.iota(jnp.int32, 16)

        def group(m, _):
            uvec = uidx_v[pl.ds(m * 16, 16)]
            ivec = iidx_v[pl.ds(m * 16, 16)]
            for sb in range(2):
                copies = []
                for j in range(8):
                    ru = uvec[sb * 8 + j]
                    ri = ivec[sb * 8 + j]
                    qu = pl.multiple_of(
                        lax.shift_left(lax.shift_right_logical(ru, 7), 7), 128)
                    qi = pl.multiple_of(
                        lax.shift_left(lax.shift_right_logical(ri, 7), 7), 128)
                    copies.append(pltpu.async_copy(
                        utab_hbm.at[:, pl.ds(qu, 128)],
                        ubuf_v.at[:, pl.ds(j * 128, 128)], usem))
                    copies.append(pltpu.async_copy(
                        itab_hbm.at[:, pl.ds(qi, 128)],
                        ibuf_v.at[:, pl.ds(j * 128, 128)], isem))
                for cp in copies:
                    cp.wait()
                for j in range(8):
                    ru = uvec[sb * 8 + j]
                    ri = ivec[sb * 8 + j]
                    lu = lanes * 0 + (j * 128 + lax.bitwise_and(ru, 127))
                    li = lanes * 0 + (j * 128 + lax.bitwise_and(ri, 127))
                    col = lanes * 0 + (m * 16 + sb * 8 + j)
                    uval = plsc.load_gather(ubuf_v, [lanes, lu])
                    plsc.store_scatter(urows_v, [lanes, col], uval)
                    ival = plsc.load_gather(ibuf_v, [lanes, li])
                    plsc.store_scatter(irows_v, [lanes, col], ival)
            return 0

        lax.fori_loop(0, bpw // 16, group, 0)
        aligned_base = pl.multiple_of(base, 128)
        pltpu.sync_copy(urows_v, uout_hbm.at[:, pl.ds(aligned_base, bpw)])
        pltpu.sync_copy(irows_v, iout_hbm.at[:, pl.ds(aligned_base, bpw)])

    return gather_kernel


_BB = 2048


def _tc_body(u_ref, i_ref, w1ut_ref, w1it_ref, b1_ref, w2t_ref, b2_ref,
             w3_ref, c0_ref, out_ref):
    uT = u_ref[...]
    iT = i_ref[...]
    inter = jnp.sum(uT * iT, axis=0)
    h1 = jnp.dot(w1ut_ref[...], uT, preferred_element_type=jnp.float32)
    h1 = h1 + jnp.dot(w1it_ref[...], iT, preferred_element_type=jnp.float32)
    h1 = jnp.maximum(h1 + b1_ref[...], 0.0)
    h2 = jnp.dot(w2t_ref[...], h1, preferred_element_type=jnp.float32)
    h2 = jnp.maximum(h2 + b2_ref[...], 0.0)
    deep = jnp.sum(h2 * w3_ref[...], axis=0)
    out_ref[...] = inter + deep + c0_ref[0]


def _tc_mlp(uT, iT, w1ut, w1it, b1col, w2t, b2col, w3col, c0):
    rep = lambda shape: pl.BlockSpec(shape, lambda i: (0,) * len(shape))
    return pl.pallas_call(
        _tc_body,
        grid=(B // _BB,),
        in_specs=[
            pl.BlockSpec((D, _BB), lambda i: (0, i)),
            pl.BlockSpec((D, _BB), lambda i: (0, i)),
            rep((H1, D)),
            rep((H1, D)),
            rep((H1, 1)),
            rep((H2, H1)),
            rep((H2, 1)),
            rep((H2, 1)),
            pl.BlockSpec(memory_space=pltpu.SMEM),
        ],
        out_specs=pl.BlockSpec((_BB,), lambda i: (i,)),
        out_shape=jax.ShapeDtypeStruct((B,), jnp.float32),
    )(uT, iT, w1ut, w1it, b1col, w2t, b2col, w3col, c0)


def kernel(user_id, item_id, user_table, item_table, fm_bias, W1, b1, W2, b2,
           W3, b3):
    uid = user_id.astype(jnp.int32)
    iid = item_id.astype(jnp.int32)
    uT, iT = _sc_gather()(uid, iid, user_table.T, item_table.T)
    c0 = fm_bias + b3
    W1t = W1.T
    return _tc_mlp(uT, iT, W1t[:, :D], W1t[:, D:], b1.reshape(H1, 1), W2.T,
                   b2.reshape(H2, 1), W3, c0)


# speedup vs baseline: 5.2687x; 1.0667x over previous
import functools

import jax
import jax.numpy as jnp
from jax import lax
from jax.experimental import pallas as pl
from jax.experimental.pallas import tpu as pltpu
from jax.experimental.pallas import tpu_sc as plsc

B = 16384
D = 16
H1 = 64
H2 = 32


@functools.cache
def _sc_gather():
    info = plsc.get_sparse_core_info()
    nw = info.num_cores * info.num_subcores
    bpw = B // nw
    mesh = plsc.VectorSubcoreMesh(core_axis_name="c", subcore_axis_name="s")

    @functools.partial(
        pl.kernel,
        out_type=(
            jax.ShapeDtypeStruct((D, B), jnp.float32),
            jax.ShapeDtypeStruct((D, B), jnp.float32),
        ),
        mesh=mesh,
        compiler_params=pltpu.CompilerParams(use_tc_tiling_on_sc=True,
                                             needs_layout_passes=False),
        scratch_types=[
            pltpu.VMEM((bpw,), jnp.int32),
            pltpu.VMEM((bpw,), jnp.int32),
            pltpu.VMEM((D, 16 * 128), jnp.float32),
            pltpu.VMEM((D, 16 * 128), jnp.float32),
            pltpu.VMEM((D, bpw), jnp.float32),
            pltpu.VMEM((D, bpw), jnp.float32),
            pltpu.SemaphoreType.DMA,
            pltpu.SemaphoreType.DMA,
        ],
    )
    def gather_kernel(uid_hbm, iid_hbm, utab_hbm, itab_hbm, uout_hbm, iout_hbm,
                      uidx_v, iidx_v, ubuf_v, ibuf_v, urows_v, irows_v,
                      usem, isem):
        wid = lax.axis_index("s") * info.num_cores + lax.axis_index("c")
        base = wid * bpw
        pltpu.sync_copy(uid_hbm.at[pl.ds(base, bpw)], uidx_v)
        pltpu.sync_copy(iid_hbm.at[pl.ds(base, bpw)], iidx_v)
        lanes = lax.iota(jnp.int32, 16)

        def group(m, _):
            uvec = uidx_v[pl.ds(m * 16, 16)]
            ivec = iidx_v[pl.ds(m * 16, 16)]
            copies = []
            for j in range(16):
                ru = uvec[j]
                ri = ivec[j]
                qu = pl.multiple_of(
                    lax.shift_left(lax.shift_right_logical(ru, 7), 7), 128)
                qi = pl.multiple_of(
                    lax.shift_left(lax.shift_right_logical(ri, 7), 7), 128)
                copies.append(pltpu.async_copy(
                    utab_hbm.at[:, pl.ds(qu, 128)],
                    ubuf_v.at[:, pl.ds(j * 128, 128)], usem))
                copies.append(pltpu.async_copy(
                    itab_hbm.at[:, pl.ds(qi, 128)],
                    ibuf_v.at[:, pl.ds(j * 128, 128)], isem))
            for cp in copies:
                cp.wait()
            for j in range(16):
                ru = uvec[j]
                ri = ivec[j]
                lu = lanes * 0 + (j * 128 + lax.bitwise_and(ru, 127))
                li = lanes * 0 + (j * 128 + lax.bitwise_and(ri, 127))
                col = lanes * 0 + (m * 16 + j)
                uval = plsc.load_gather(ubuf_v, [lanes, lu])
                plsc.store_scatter(urows_v, [lanes, col], uval)
                ival = plsc.load_gather(ibuf_v, [lanes, li])
                plsc.store_scatter(irows_v, [lanes, col], ival)
            return 0

        lax.fori_loop(0, bpw // 16, group, 0)
        aligned_base = pl.multiple_of(base, 128)
        pltpu.sync_copy(urows_v, uout_hbm.at[:, pl.ds(aligned_base, bpw)])
        pltpu.sync_copy(irows_v, iout_hbm.at[:, pl.ds(aligned_base, bpw)])

    return gather_kernel


_BB = 2048


def _tc_body(u_ref, i_ref, w1ut_ref, w1it_ref, b1_ref, w2t_ref, b2_ref,
             w3_ref, c0_ref, out_ref):
    uT = u_ref[...]
    iT = i_ref[...]
    inter = jnp.sum(uT * iT, axis=0)
    h1 = jnp.dot(w1ut_ref[...], uT, preferred_element_type=jnp.float32)
    h1 = h1 + jnp.dot(w1it_ref[...], iT, preferred_element_type=jnp.float32)
    h1 = jnp.maximum(h1 + b1_ref[...], 0.0)
    h2 = jnp.dot(w2t_ref[...], h1, preferred_element_type=jnp.float32)
    h2 = jnp.maximum(h2 + b2_ref[...], 0.0)
    deep = jnp.sum(h2 * w3_ref[...], axis=0)
    out_ref[...] = inter + deep + c0_ref[0]


def _tc_mlp(uT, iT, w1ut, w1it, b1col, w2t, b2col, w3col, c0):
    rep = lambda shape: pl.BlockSpec(shape, lambda i: (0,) * len(shape))
    return pl.pallas_call(
        _tc_body,
        grid=(B // _BB,),
        in_specs=[
            pl.BlockSpec((D, _BB), lambda i: (0, i)),
            pl.BlockSpec((D, _BB), lambda i: (0, i)),
            rep((H1, D)),
            rep((H1, D)),
            rep((H1, 1)),
            rep((H2, H1)),
            rep((H2, 1)),
            rep((H2, 1)),
            pl.BlockSpec(memory_space=pltpu.SMEM),
        ],
        out_specs=pl.BlockSpec((_BB,), lambda i: (i,)),
        out_shape=jax.ShapeDtypeStruct((B,), jnp.float32),
    )(uT, iT, w1ut, w1it, b1col, w2t, b2col, w3col, c0)


def kernel(user_id, item_id, user_table, item_table, fm_bias, W1, b1, W2, b2,
           W3, b3):
    uid = user_id.astype(jnp.int32)
    iid = item_id.astype(jnp.int32)
    uT, iT = _sc_gather()(uid, iid, user_table.T, item_table.T)
    c0 = fm_bias + b3
    W1t = W1.T
    return _tc_mlp(uT, iT, W1t[:, :D], W1t[:, D:], b1.reshape(H1, 1), W2.T,
                   b2.reshape(H2, 1), W3, c0)
